# Initial kernel scaffold; baseline (speedup 1.0000x reference)
import jax
import jax.numpy as jnp
from jax.experimental import pallas as pl


def kernel(z_e, emb_table):
    raise NotImplementedError("write your pallas kernel here")


# speedup vs baseline: 1.3969x; 1.3969x over previous
import functools

import jax
import jax.numpy as jnp
from jax import lax
from jax.experimental import pallas as pl
from jax.experimental.pallas import tpu as pltpu

_VOCAB = 1024
_DIM = 64


def _nn_body(z_ref, emb_ref, w_ref, wemb_ref):
    z = z_ref[0]
    emb = emb_ref[...]
    ab = lax.dot_general(z, emb, (((1,), (1,)), ((), ())),
                         preferred_element_type=jnp.float32)
    a2 = jnp.sum(z * z, axis=1, keepdims=True)
    b2 = jnp.sum(emb * emb, axis=1)[None, :]
    d2 = a2 + b2 - 2.0 * ab
    d = jnp.sqrt(jnp.maximum(d2, 0.0))
    m = jnp.min(d, axis=1, keepdims=True)
    col = lax.broadcasted_iota(jnp.int32, d.shape, 1)
    w = jnp.min(jnp.where(d == m, col, _VOCAB), axis=1)
    w_ref[0, 0] = w
    onehot = (w[:, None] == col).astype(jnp.float32)
    wemb_ref[0] = lax.dot_general(onehot, emb, (((1,), (0,)), ((), ())),
                                  preferred_element_type=jnp.float32)


@functools.partial(jax.jit, static_argnames=("tb", "interpret"))
def _nn_call(z_flat, emb_table, tb=1152, interpret=False):
    n = z_flat.shape[0]
    nb = n // tb
    zb = z_flat.reshape(nb, tb, _DIM)
    w, wemb = pl.pallas_call(
        _nn_body,
        grid=(nb,),
        in_specs=[
            pl.BlockSpec((1, tb, _DIM), lambda i: (i, 0, 0)),
            pl.BlockSpec((_VOCAB, _DIM), lambda i: (0, 0)),
        ],
        out_specs=[
            pl.BlockSpec((1, 1, tb), lambda i: (i, 0, 0)),
            pl.BlockSpec((1, tb, _DIM), lambda i: (i, 0, 0)),
        ],
        out_shape=[
            jax.ShapeDtypeStruct((nb, 1, tb), jnp.int32),
            jax.ShapeDtypeStruct((nb, tb, _DIM), jnp.float32),
        ],
        interpret=interpret,
    )(zb, emb_table)
    return w.reshape(n), wemb.reshape(n, _DIM)


def kernel(z_e, emb_table):
    bs, t, d = z_e.shape
    z_flat = z_e.reshape(bs * t, d)
    w, wemb = _nn_call(z_flat, emb_table)
    return w.reshape(bs, t), wemb.reshape(bs, t, d)


# speedup vs baseline: 1.4252x; 1.0203x over previous
import functools

import jax
import jax.numpy as jnp
from jax import lax
from jax.experimental import pallas as pl
from jax.experimental.pallas import tpu as pltpu

_VOCAB = 1024
_DIM = 64


def _nn_body(z_ref, emb_ref, w_ref, wemb_ref):
    z = z_ref[0]
    emb = emb_ref[...]
    ab = lax.dot_general(z, emb, (((1,), (1,)), ((), ())),
                         preferred_element_type=jnp.float32)
    a2 = jnp.sum(z * z, axis=1, keepdims=True)
    b2 = jnp.sum(emb * emb, axis=1)[None, :]
    d2 = a2 + b2 - 2.0 * ab
    d = jnp.sqrt(jnp.maximum(d2, 0.0))
    m = jnp.min(d, axis=1, keepdims=True)
    col = lax.broadcasted_iota(jnp.int32, d.shape, 1)
    w = jnp.min(jnp.where(d == m, col, _VOCAB), axis=1)
    w_ref[0, 0] = w
    onehot = (w[:, None] == col).astype(jnp.float32)
    wemb_ref[0] = lax.dot_general(onehot, emb, (((1,), (0,)), ((), ())),
                                  preferred_element_type=jnp.float32)


@functools.partial(jax.jit, static_argnames=("tb", "interpret"))
def _nn_call(z_flat, emb_table, tb=2304, interpret=False):
    n = z_flat.shape[0]
    nb = n // tb
    zb = z_flat.reshape(nb, tb, _DIM)
    w, wemb = pl.pallas_call(
        _nn_body,
        grid=(nb,),
        in_specs=[
            pl.BlockSpec((1, tb, _DIM), lambda i: (i, 0, 0)),
            pl.BlockSpec((_VOCAB, _DIM), lambda i: (0, 0)),
        ],
        out_specs=[
            pl.BlockSpec((1, 1, tb), lambda i: (i, 0, 0)),
            pl.BlockSpec((1, tb, _DIM), lambda i: (i, 0, 0)),
        ],
        out_shape=[
            jax.ShapeDtypeStruct((nb, 1, tb), jnp.int32),
            jax.ShapeDtypeStruct((nb, tb, _DIM), jnp.float32),
        ],
        interpret=interpret,
    )(zb, emb_table)
    return w.reshape(n), wemb.reshape(n, _DIM)


def kernel(z_e, emb_table):
    bs, t, d = z_e.shape
    z_flat = z_e.reshape(bs * t, d)
    w, wemb = _nn_call(z_flat, emb_table)
    return w.reshape(bs, t), wemb.reshape(bs, t, d)


# speedup vs baseline: 1.5121x; 1.0610x over previous
import functools

import jax
import jax.numpy as jnp
from jax import lax
from jax.experimental import pallas as pl
from jax.experimental.pallas import tpu as pltpu

_VOCAB = 1024
_DIM = 64


def _nn_body(z_ref, emb_ref, w_ref, wemb_ref):
    z = z_ref[0]
    emb = emb_ref[...]
    ab2 = lax.dot_general(z, emb * -2.0, (((1,), (1,)), ((), ())),
                          preferred_element_type=jnp.float32)
    a2 = jnp.sum(z * z, axis=1, keepdims=True)
    b2 = jnp.sum(emb * emb, axis=1)[None, :]
    s = a2 + b2
    run_v = None
    run_i = None
    for j in range(_VOCAB // 128):
        sl = slice(j * 128, (j + 1) * 128)
        d2 = s[:, sl] + ab2[:, sl]
        d = jnp.sqrt(jnp.maximum(d2, 0.0))
        col = lax.broadcasted_iota(jnp.int32, d.shape, 1) + j * 128
        if run_v is None:
            run_v, run_i = d, col
        else:
            run_i = jnp.where(d < run_v, col, run_i)
            run_v = jnp.minimum(run_v, d)
    m = jnp.min(run_v, axis=1, keepdims=True)
    w = jnp.min(jnp.where(run_v == m, run_i, _VOCAB), axis=1)
    w_ref[0, 0] = w
    col = lax.broadcasted_iota(jnp.int32, (z.shape[0], _VOCAB), 1)
    onehot = (w[:, None] == col).astype(jnp.float32)
    wemb_ref[0] = lax.dot_general(onehot, emb, (((1,), (0,)), ((), ())),
                                  preferred_element_type=jnp.float32)


@functools.partial(jax.jit, static_argnames=("tb", "interpret"))
def _nn_call(z_flat, emb_table, tb=2304, interpret=False):
    n = z_flat.shape[0]
    nb = n // tb
    zb = z_flat.reshape(nb, tb, _DIM)
    w, wemb = pl.pallas_call(
        _nn_body,
        grid=(nb,),
        in_specs=[
            pl.BlockSpec((1, tb, _DIM), lambda i: (i, 0, 0)),
            pl.BlockSpec((_VOCAB, _DIM), lambda i: (0, 0)),
        ],
        out_specs=[
            pl.BlockSpec((1, 1, tb), lambda i: (i, 0, 0)),
            pl.BlockSpec((1, tb, _DIM), lambda i: (i, 0, 0)),
        ],
        out_shape=[
            jax.ShapeDtypeStruct((nb, 1, tb), jnp.int32),
            jax.ShapeDtypeStruct((nb, tb, _DIM), jnp.float32),
        ],
        interpret=interpret,
    )(zb, emb_table)
    return w.reshape(n), wemb.reshape(n, _DIM)


def kernel(z_e, emb_table):
    bs, t, d = z_e.shape
    z_flat = z_e.reshape(bs * t, d)
    w, wemb = _nn_call(z_flat, emb_table)
    return w.reshape(bs, t), wemb.reshape(bs, t, d)
